# Initial kernel scaffold; baseline (speedup 1.0000x reference)
import jax
import jax.numpy as jnp
from jax.experimental import pallas as pl


def kernel(node_features, edge_index, W1, b1, W2, b2, W3, b3, W4, b4):
    raise NotImplementedError("write your pallas kernel here")


# speedup vs baseline: 2.8731x; 2.8731x over previous
import jax
import jax.numpy as jnp
from jax import lax
from jax.experimental import pallas as pl
from jax.experimental.pallas import tpu as pltpu
from jax.experimental.pallas import tpu_sc as plsc

N = 10000
D = 128
DH = 64
E = 320000
DOUT = 64

NC, NS = 2, 16
C = 128
NCHUNK = 2528
CPT = NCHUNK // NS
EPAD = NCHUNK * C
NROWS = 10112
RPT = NROWS // NS

_TC_BLK = 2000


def _sc_edge_body(p0_hbm, p1_hbm, q0_hbm, q1_hbm, srcm, tgtm, z64, z16, ones16,
                  hs0_out, hs1_out, dg_out,
                  src_v, tgt_v, p_v, q_v, ones_v, hsh, hdg, sem_p, sem_q):
    c = lax.axis_index("c")
    s = lax.axis_index("s")
    base_row = s * RPT

    pltpu.sync_copy(z64, hsh.at[pl.ds(base_row, RPT)])
    pltpu.sync_copy(z16, hdg.at[pl.ds(base_row, RPT)])
    pltpu.sync_copy(ones16, ones_v)
    plsc.subcore_barrier()

    def chunk_loop(p_src, q_src, deg_parity):
        def body(j, carry):
            row = s * CPT + j
            pltpu.sync_copy(srcm.at[row], src_v)
            pltpu.sync_copy(tgtm.at[row], tgt_v)
            cp_p = pltpu.async_copy(p_src.at[src_v], p_v, sem_p)
            cp_q = pltpu.async_copy(q_src.at[tgt_v], q_v, sem_q)
            cp_p.wait()
            cp_q.wait()

            def relu_row(i, carry2):
                for jj in range(DH // 16):
                    sl = pl.ds(jj * 16, 16)
                    p_v[i, sl] = jnp.maximum(p_v[i, sl] + q_v[i, sl], 0.0)
                return carry2
            lax.fori_loop(0, C, relu_row, 0, unroll=2)

            pltpu.sync_copy(p_v, hsh.at[tgt_v], add=True)

            @pl.when(lax.rem(row, 2) == deg_parity)
            def _():
                pltpu.sync_copy(ones_v, hdg.at[tgt_v], add=True)
            return carry
        lax.fori_loop(0, CPT, body, 0)

    @pl.when(c == 0)
    def _():
        chunk_loop(p0_hbm, q0_hbm, 0)

    @pl.when(c != 0)
    def _():
        chunk_loop(p1_hbm, q1_hbm, 1)

    plsc.subcore_barrier()

    sl_rows = pl.ds(base_row, RPT)

    @pl.when(c == 0)
    def _():
        pltpu.sync_copy(hsh.at[sl_rows], hs0_out.at[sl_rows])

    @pl.when(c != 0)
    def _():
        pltpu.sync_copy(hsh.at[sl_rows], hs1_out.at[sl_rows])

    pltpu.sync_copy(hdg.at[sl_rows], dg_out.at[pl.ds(c * NROWS + base_row, RPT)])


_sc_edge = pl.kernel(
    _sc_edge_body,
    out_type=(
        jax.ShapeDtypeStruct((NROWS, DH), jnp.float32),
        jax.ShapeDtypeStruct((NROWS, DH), jnp.float32),
        jax.ShapeDtypeStruct((NC * NROWS, 16), jnp.float32),
    ),
    mesh=plsc.VectorSubcoreMesh(
        core_axis_name="c", subcore_axis_name="s",
        num_cores=NC, num_subcores=NS,
    ),
    compiler_params=pltpu.CompilerParams(use_tc_tiling_on_sc=False),
    scratch_types=[
        pltpu.VMEM((C,), jnp.int32),
        pltpu.VMEM((C,), jnp.int32),
        pltpu.VMEM((C, DH), jnp.float32),
        pltpu.VMEM((C, DH), jnp.float32),
        pltpu.VMEM((C, 16), jnp.float32),
        pltpu.VMEM_SHARED((NROWS, DH), jnp.float32),
        pltpu.VMEM_SHARED((NROWS, 16), jnp.float32),
        pltpu.SemaphoreType.DMA,
        pltpu.SemaphoreType.DMA,
    ],
)


def _tc1_body(x_ref, wa0_ref, wa1_ref, wb0_ref, wb1_ref, b10_ref, b11_ref,
              p0_ref, p1_ref, q0_ref, q1_ref):
    x = x_ref[...]
    p0_ref[...] = jnp.dot(x, wa0_ref[...], preferred_element_type=jnp.float32)
    p1_ref[...] = jnp.dot(x, wa1_ref[...], preferred_element_type=jnp.float32)
    q0_ref[...] = (jnp.dot(x, wb0_ref[...], preferred_element_type=jnp.float32)
                   + b10_ref[...])
    q1_ref[...] = (jnp.dot(x, wb1_ref[...], preferred_element_type=jnp.float32)
                   + b11_ref[...])


def _tc2_body(h0_ref, h1_ref, d0_ref, d1_ref, x_ref,
              w2t_ref, w2b_ref, w3a_ref, w3b_ref, b3_ref, w4_ref, b2_ref,
              b4_ref, out_ref):
    dv = d0_ref[...] + d1_ref[...]
    deg = dv[:, :1]
    agg = (jnp.dot(h0_ref[...], w2t_ref[...], preferred_element_type=jnp.float32)
           + jnp.dot(h1_ref[...], w2b_ref[...], preferred_element_type=jnp.float32)
           + deg * b2_ref[...])
    u = jnp.maximum(
        jnp.dot(x_ref[...], w3a_ref[...], preferred_element_type=jnp.float32)
        + jnp.dot(agg, w3b_ref[...], preferred_element_type=jnp.float32)
        + b3_ref[...], 0.0)
    out_ref[...] = (jnp.dot(u, w4_ref[...], preferred_element_type=jnp.float32)
                    + b4_ref[...])


def _row_blk(i):
    return (i, 0)


def _full(i):
    return (0, 0)


_tc1 = pl.pallas_call(
    _tc1_body,
    grid=(N // _TC_BLK,),
    in_specs=[
        pl.BlockSpec((_TC_BLK, D), _row_blk),
        pl.BlockSpec((D, DH), _full),
        pl.BlockSpec((D, DH), _full),
        pl.BlockSpec((D, DH), _full),
        pl.BlockSpec((D, DH), _full),
        pl.BlockSpec((1, DH), _full),
        pl.BlockSpec((1, DH), _full),
    ],
    out_specs=[
        pl.BlockSpec((_TC_BLK, DH), _row_blk),
        pl.BlockSpec((_TC_BLK, DH), _row_blk),
        pl.BlockSpec((_TC_BLK, DH), _row_blk),
        pl.BlockSpec((_TC_BLK, DH), _row_blk),
    ],
    out_shape=[
        jax.ShapeDtypeStruct((N, DH), jnp.float32),
        jax.ShapeDtypeStruct((N, DH), jnp.float32),
        jax.ShapeDtypeStruct((N, DH), jnp.float32),
        jax.ShapeDtypeStruct((N, DH), jnp.float32),
    ],
)

_tc2 = pl.pallas_call(
    _tc2_body,
    grid=(N // _TC_BLK,),
    in_specs=[
        pl.BlockSpec((_TC_BLK, DH), _row_blk),
        pl.BlockSpec((_TC_BLK, DH), _row_blk),
        pl.BlockSpec((_TC_BLK, 16), _row_blk),
        pl.BlockSpec((_TC_BLK, 16), _row_blk),
        pl.BlockSpec((_TC_BLK, D), _row_blk),
        pl.BlockSpec((DH, D), _full),
        pl.BlockSpec((DH, D), _full),
        pl.BlockSpec((D, D), _full),
        pl.BlockSpec((D, D), _full),
        pl.BlockSpec((1, D), _full),
        pl.BlockSpec((D, DOUT), _full),
        pl.BlockSpec((1, D), _full),
        pl.BlockSpec((1, DOUT), _full),
    ],
    out_specs=pl.BlockSpec((_TC_BLK, DOUT), _row_blk),
    out_shape=jax.ShapeDtypeStruct((N, DOUT), jnp.float32),
)


@jax.jit
def _run(x, src, tgt, W1, b1, W2, b2, W3, b3, W4, b4):
    pad = EPAD - E
    srcm = jnp.concatenate([src, jnp.zeros((pad,), jnp.int32)]).reshape(NCHUNK, C)
    tgtm = jnp.concatenate([tgt, jnp.full((pad,), N, jnp.int32)]).reshape(NCHUNK, C)

    w1t = W1.T
    wa, wb = w1t[:D], w1t[D:]
    p0, p1, q0, q1 = _tc1(x, wa[:, :DH], wa[:, DH:], wb[:, :DH], wb[:, DH:],
                          b1[:DH].reshape(1, DH), b1[DH:].reshape(1, DH))

    z64 = jnp.zeros((RPT, DH), jnp.float32)
    z16 = jnp.zeros((RPT, 16), jnp.float32)
    ones16 = jnp.ones((C, 16), jnp.float32)
    hs0, hs1, dg = _sc_edge(p0, p1, q0, q1, srcm, tgtm, z64, z16, ones16)

    w2t = W2.T
    w3t = W3.T
    out = _tc2(hs0, hs1, dg[:NROWS], dg[NROWS:], x,
               w2t[:DH], w2t[DH:], w3t[:D], w3t[D:], b3.reshape(1, D),
               W4.T, b2.reshape(1, D), b4.reshape(1, DOUT))
    return out


def kernel(node_features, edge_index, W1, b1, W2, b2, W3, b3, W4, b4):
    ei = edge_index.astype(jnp.int32)
    return _run(node_features, ei[0], ei[1], W1, b1, W2, b2, W3, b3, W4, b4)


# speedup vs baseline: 4.2609x; 1.4830x over previous
import jax
import jax.numpy as jnp
from jax import lax
from jax.experimental import pallas as pl
from jax.experimental.pallas import tpu as pltpu
from jax.experimental.pallas import tpu_sc as plsc

N = 10000
D = 128
DH = 64
E = 320000
DOUT = 64

NC, NS = 2, 16
C = 128
NCHUNK = 2560
CPT = NCHUNK // NS
EPAD = NCHUNK * C
NROWS = 10112
RPT = NROWS // NS

_TC_BLK = 2000


def _sc_edge_body(p0_hbm, p1_hbm, q0_hbm, q1_hbm, srcm, tgtm, z64, z16, ones16,
                  hs0_out, hs1_out, dg_out,
                  src_all, tgt_all, p_bufs, q_bufs, ones_v, hsh, hdg,
                  sem_p0, sem_p1, sem_q0, sem_q1):
    c = lax.axis_index("c")
    s = lax.axis_index("s")
    base_row = s * RPT

    pltpu.sync_copy(z64, hsh.at[pl.ds(base_row, RPT)])
    pltpu.sync_copy(z16, hdg.at[pl.ds(base_row, RPT)])
    pltpu.sync_copy(ones16, ones_v)
    pltpu.sync_copy(srcm.at[pl.ds(s * CPT, CPT)], src_all)
    pltpu.sync_copy(tgtm.at[pl.ds(s * CPT, CPT)], tgt_all)
    plsc.subcore_barrier()

    sems_p = (sem_p0, sem_p1)
    sems_q = (sem_q0, sem_q1)

    def chunk_loop(p_src, q_src, deg_parity):
        pltpu.async_copy(p_src.at[src_all.at[0]], p_bufs.at[0], sem_p0)
        pltpu.async_copy(q_src.at[tgt_all.at[0]], q_bufs.at[0], sem_q0)

        def outer(j, carry):
            for b in range(2):
                jj = 2 * j + b
                p_v = p_bufs.at[b]
                q_v = q_bufs.at[b]
                nb = 1 - b
                pltpu.make_async_copy(
                    p_src.at[src_all.at[jj]], p_v, sems_p[b]).wait()
                pltpu.make_async_copy(
                    q_src.at[tgt_all.at[jj]], q_v, sems_q[b]).wait()

                @pl.when(jj + 1 < CPT)
                def _():
                    pltpu.async_copy(
                        p_src.at[src_all.at[jj + 1]], p_bufs.at[nb], sems_p[nb])
                    pltpu.async_copy(
                        q_src.at[tgt_all.at[jj + 1]], q_bufs.at[nb], sems_q[nb])

                def relu_row(i, carry2):
                    for cc in range(DH // 16):
                        sl = pl.ds(cc * 16, 16)
                        p_v[i, sl] = jnp.maximum(p_v[i, sl] + q_v[i, sl], 0.0)
                    return carry2
                lax.fori_loop(0, C, relu_row, 0, unroll=4)

                tgt_idx = tgt_all.at[jj]
                pltpu.sync_copy(p_v, hsh.at[tgt_idx], add=True)
                if b == 0:
                    is_deg = (deg_parity == 0)
                else:
                    is_deg = (deg_parity == 1)
                if is_deg:
                    pltpu.sync_copy(ones_v, hdg.at[tgt_idx], add=True)
            return carry
        lax.fori_loop(0, CPT // 2, outer, 0)

    @pl.when(c == 0)
    def _():
        chunk_loop(p0_hbm, q0_hbm, 0)

    @pl.when(c != 0)
    def _():
        chunk_loop(p1_hbm, q1_hbm, 1)

    plsc.subcore_barrier()

    sl_rows = pl.ds(base_row, RPT)

    @pl.when(c == 0)
    def _():
        pltpu.sync_copy(hsh.at[sl_rows], hs0_out.at[sl_rows])

    @pl.when(c != 0)
    def _():
        pltpu.sync_copy(hsh.at[sl_rows], hs1_out.at[sl_rows])

    pltpu.sync_copy(hdg.at[sl_rows], dg_out.at[pl.ds(c * NROWS + base_row, RPT)])


_sc_edge = pl.kernel(
    _sc_edge_body,
    out_type=(
        jax.ShapeDtypeStruct((NROWS, DH), jnp.float32),
        jax.ShapeDtypeStruct((NROWS, DH), jnp.float32),
        jax.ShapeDtypeStruct((NC * NROWS, 16), jnp.float32),
    ),
    mesh=plsc.VectorSubcoreMesh(
        core_axis_name="c", subcore_axis_name="s",
        num_cores=NC, num_subcores=NS,
    ),
    compiler_params=pltpu.CompilerParams(use_tc_tiling_on_sc=False),
    scratch_types=[
        pltpu.VMEM((CPT, C), jnp.int32),
        pltpu.VMEM((CPT, C), jnp.int32),
        pltpu.VMEM((2, C, DH), jnp.float32),
        pltpu.VMEM((2, C, DH), jnp.float32),
        pltpu.VMEM((C, 16), jnp.float32),
        pltpu.VMEM_SHARED((NROWS, DH), jnp.float32),
        pltpu.VMEM_SHARED((NROWS, 16), jnp.float32),
        pltpu.SemaphoreType.DMA,
        pltpu.SemaphoreType.DMA,
        pltpu.SemaphoreType.DMA,
        pltpu.SemaphoreType.DMA,
    ],
)


def _tc1_body(x_ref, wa0_ref, wa1_ref, wb0_ref, wb1_ref, b10_ref, b11_ref,
              p0_ref, p1_ref, q0_ref, q1_ref):
    x = x_ref[...]
    p0_ref[...] = jnp.dot(x, wa0_ref[...], preferred_element_type=jnp.float32)
    p1_ref[...] = jnp.dot(x, wa1_ref[...], preferred_element_type=jnp.float32)
    q0_ref[...] = (jnp.dot(x, wb0_ref[...], preferred_element_type=jnp.float32)
                   + b10_ref[...])
    q1_ref[...] = (jnp.dot(x, wb1_ref[...], preferred_element_type=jnp.float32)
                   + b11_ref[...])


def _tc2_body(h0_ref, h1_ref, d0_ref, d1_ref, x_ref,
              w2t_ref, w2b_ref, w3a_ref, w3b_ref, b3_ref, w4_ref, b2_ref,
              b4_ref, out_ref):
    dv = d0_ref[...] + d1_ref[...]
    deg = dv[:, :1]
    agg = (jnp.dot(h0_ref[...], w2t_ref[...], preferred_element_type=jnp.float32)
           + jnp.dot(h1_ref[...], w2b_ref[...], preferred_element_type=jnp.float32)
           + deg * b2_ref[...])
    u = jnp.maximum(
        jnp.dot(x_ref[...], w3a_ref[...], preferred_element_type=jnp.float32)
        + jnp.dot(agg, w3b_ref[...], preferred_element_type=jnp.float32)
        + b3_ref[...], 0.0)
    out_ref[...] = (jnp.dot(u, w4_ref[...], preferred_element_type=jnp.float32)
                    + b4_ref[...])


def _row_blk(i):
    return (i, 0)


def _full(i):
    return (0, 0)


_tc1 = pl.pallas_call(
    _tc1_body,
    grid=(N // _TC_BLK,),
    in_specs=[
        pl.BlockSpec((_TC_BLK, D), _row_blk),
        pl.BlockSpec((D, DH), _full),
        pl.BlockSpec((D, DH), _full),
        pl.BlockSpec((D, DH), _full),
        pl.BlockSpec((D, DH), _full),
        pl.BlockSpec((1, DH), _full),
        pl.BlockSpec((1, DH), _full),
    ],
    out_specs=[
        pl.BlockSpec((_TC_BLK, DH), _row_blk),
        pl.BlockSpec((_TC_BLK, DH), _row_blk),
        pl.BlockSpec((_TC_BLK, DH), _row_blk),
        pl.BlockSpec((_TC_BLK, DH), _row_blk),
    ],
    out_shape=[
        jax.ShapeDtypeStruct((N, DH), jnp.float32),
        jax.ShapeDtypeStruct((N, DH), jnp.float32),
        jax.ShapeDtypeStruct((N, DH), jnp.float32),
        jax.ShapeDtypeStruct((N, DH), jnp.float32),
    ],
)

_tc2 = pl.pallas_call(
    _tc2_body,
    grid=(N // _TC_BLK,),
    in_specs=[
        pl.BlockSpec((_TC_BLK, DH), _row_blk),
        pl.BlockSpec((_TC_BLK, DH), _row_blk),
        pl.BlockSpec((_TC_BLK, 16), _row_blk),
        pl.BlockSpec((_TC_BLK, 16), _row_blk),
        pl.BlockSpec((_TC_BLK, D), _row_blk),
        pl.BlockSpec((DH, D), _full),
        pl.BlockSpec((DH, D), _full),
        pl.BlockSpec((D, D), _full),
        pl.BlockSpec((D, D), _full),
        pl.BlockSpec((1, D), _full),
        pl.BlockSpec((D, DOUT), _full),
        pl.BlockSpec((1, D), _full),
        pl.BlockSpec((1, DOUT), _full),
    ],
    out_specs=pl.BlockSpec((_TC_BLK, DOUT), _row_blk),
    out_shape=jax.ShapeDtypeStruct((N, DOUT), jnp.float32),
)


@jax.jit
def _run(x, src, tgt, W1, b1, W2, b2, W3, b3, W4, b4):
    pad = EPAD - E
    srcm = jnp.concatenate([src, jnp.zeros((pad,), jnp.int32)]).reshape(NCHUNK, C)
    tgtm = jnp.concatenate([tgt, jnp.full((pad,), N, jnp.int32)]).reshape(NCHUNK, C)

    w1t = W1.T
    wa, wb = w1t[:D], w1t[D:]
    p0, p1, q0, q1 = _tc1(x, wa[:, :DH], wa[:, DH:], wb[:, :DH], wb[:, DH:],
                          b1[:DH].reshape(1, DH), b1[DH:].reshape(1, DH))

    z64 = jnp.zeros((RPT, DH), jnp.float32)
    z16 = jnp.zeros((RPT, 16), jnp.float32)
    ones16 = jnp.ones((C, 16), jnp.float32)
    hs0, hs1, dg = _sc_edge(p0, p1, q0, q1, srcm, tgtm, z64, z16, ones16)

    w2t = W2.T
    w3t = W3.T
    out = _tc2(hs0, hs1, dg[:NROWS], dg[NROWS:], x,
               w2t[:DH], w2t[DH:], w3t[:D], w3t[D:], b3.reshape(1, D),
               W4.T, b2.reshape(1, D), b4.reshape(1, DOUT))
    return out


def kernel(node_features, edge_index, W1, b1, W2, b2, W3, b3, W4, b4):
    ei = edge_index.astype(jnp.int32)
    return _run(node_features, ei[0], ei[1], W1, b1, W2, b2, W3, b3, W4, b4)
